# baseline (speedup 1.0000x reference)
import functools

import jax
import jax.numpy as jnp
from jax import lax
from jax.experimental import pallas as pl
from jax.experimental.pallas import tpu as pltpu
from jax.experimental.pallas import tpu_sc as plsc

VOCAB = 1000000
DIM = 32
B = 4096
L = 50
NC = 2
NS = 16
NW = NC * NS
CH = B // NW
STEPS = L

_mesh = plsc.VectorSubcoreMesh(core_axis_name="c", subcore_axis_name="s")


@functools.partial(
    pl.kernel,
    out_type=jax.ShapeDtypeStruct((L, DIM, 4, B), jnp.float32),
    mesh=_mesh,
    scratch_types=[
        [pltpu.VMEM((CH,), jnp.int32) for _ in range(2)],
        [pltpu.VMEM((DIM, CH), jnp.int32) for _ in range(2)],
        [pltpu.VMEM((DIM, 4, CH), jnp.float32) for _ in range(2)],
        [pltpu.SemaphoreType.DMA for _ in range(2)],
        [pltpu.SemaphoreType.DMA for _ in range(2)],
    ],
    compiler_params=pltpu.CompilerParams(needs_layout_passes=False),
)
def _emb(x_hbm, s_hbm, vi_hbm, vj_hbm, vk_hbm, out_hbm,
         xbs, idxps, obufs, gsems, osems):
    wid = lax.axis_index("s") * NC + lax.axis_index("c")
    woff = pl.multiple_of(wid * CH, CH)
    tables = (s_hbm, vi_hbm, vj_hbm, vk_hbm)

    def fire(t, bset):
        xb, idxp, obuf = xbs[bset], idxps[bset], obufs[bset]
        pltpu.sync_copy(x_hbm.at[t, pl.ds(woff, CH)], xb)

        def per_d(d, carry):
            base = d * VOCAB
            for h in range(CH // 16):
                idxp[d, pl.ds(16 * h, 16)] = xb[pl.ds(16 * h, 16)] + base
            for q in range(4):
                pltpu.make_async_copy(
                    tables[q].at[idxp.at[d]], obuf.at[d, q],
                    gsems[bset]).start()
            return carry

        lax.fori_loop(0, DIM, per_d, 0)

    def gwait(bset):
        pltpu.make_async_copy(
            out_hbm.at[0, :, :, pl.ds(0, CH)], obufs[bset],
            gsems[bset]).wait()

    fire(0, 0)

    def pair(k, carry):
        for bset in range(2):
            t = 2 * k + bset
            gwait(bset)

            @pl.when(t > 0)
            def _():
                pltpu.make_async_copy(
                    obufs[1 - bset],
                    out_hbm.at[0, :, :, pl.ds(0, CH)],
                    osems[1 - bset]).wait()

            @pl.when(t + 1 < STEPS)
            def _():
                fire(t + 1, 1 - bset)

            pltpu.make_async_copy(
                obufs[bset],
                out_hbm.at[t, :, :, pl.ds(woff, CH)],
                osems[bset]).start()
        return carry

    lax.fori_loop(0, STEPS // 2, pair, 0)

    pltpu.make_async_copy(
        obufs[1], out_hbm.at[0, :, :, pl.ds(0, CH)], osems[1]).wait()


def kernel(x, scalar, vector_i, vector_j, vector_k):
    xt = jnp.transpose(x).astype(jnp.int32)
    flats = [jnp.transpose(a).reshape(-1)
             for a in (scalar, vector_i, vector_j, vector_k)]
    out = _emb(xt, *flats)
    return jnp.transpose(out, (3, 0, 1, 2))


# speedup vs baseline: 6.0060x; 6.0060x over previous
import functools

import jax
import jax.numpy as jnp
from jax import lax
from jax.experimental import pallas as pl
from jax.experimental.pallas import tpu as pltpu
from jax.experimental.pallas import tpu_sc as plsc

VOCAB = 1000000
DIM = 32
B = 4096
L = 50
N = B * L
NC = 2
NS = 16
NW = NC * NS
PER_W = N // NW
CH = 64
STEPS = PER_W // CH
ROW = DIM * 4

TBLK = 4096

_mesh = plsc.VectorSubcoreMesh(core_axis_name="c", subcore_axis_name="s")


def _tr_body(st, it, jt, kt, so, io, jo, ko):
    so[...] = jnp.transpose(st[...])
    io[...] = jnp.transpose(it[...])
    jo[...] = jnp.transpose(jt[...])
    ko[...] = jnp.transpose(kt[...])


_transpose_tables = pl.pallas_call(
    _tr_body,
    grid=(pl.cdiv(VOCAB, TBLK),),
    in_specs=[pl.BlockSpec((DIM, TBLK), lambda i: (0, i))] * 4,
    out_specs=[pl.BlockSpec((TBLK, DIM), lambda i: (i, 0))] * 4,
    out_shape=[jax.ShapeDtypeStruct((VOCAB, DIM), jnp.float32)] * 4,
)


@functools.partial(
    pl.kernel,
    out_type=jax.ShapeDtypeStruct((N, ROW), jnp.float32),
    mesh=_mesh,
    scratch_types=[
        [pltpu.VMEM((CH,), jnp.int32) for _ in range(2)],
        [[pltpu.VMEM((CH, DIM), jnp.float32) for _ in range(4)]
         for _ in range(2)],
        pltpu.VMEM((CH, ROW), jnp.float32),
        [pltpu.SemaphoreType.DMA for _ in range(2)],
        pltpu.SemaphoreType.DMA,
    ],
    compiler_params=pltpu.CompilerParams(needs_layout_passes=False),
)
def _emb(x_hbm, s_hbm, vi_hbm, vj_hbm, vk_hbm, out_hbm,
         idxbs, rbufs, obuf, gsems, osem):
    wid = lax.axis_index("s") * NC + lax.axis_index("c")
    lanes = lax.iota(jnp.int32, 16)
    tables = (s_hbm, vi_hbm, vj_hbm, vk_hbm)
    cols = [(64 * h + q) + 4 * lanes for q in range(4) for h in range(2)]

    def fire(t, bset):
        pltpu.sync_copy(x_hbm.at[pl.ds(wid * PER_W + t * CH, CH)],
                        idxbs[bset])

        def grp(g, carry):
            vec = idxbs[bset][pl.ds(g * 16, 16)]
            for l in range(16):
                v = vec[l]
                for q in range(4):
                    pltpu.make_async_copy(
                        tables[q].at[pl.ds(v, 1)],
                        rbufs[bset][q].at[pl.ds(g * 16 + l, 1)],
                        gsems[bset]).start()
            return carry

        lax.fori_loop(0, CH // 16, grp, 0)

    def gwait(bset):
        for q in range(4):
            pltpu.make_async_copy(
                tables[q].at[pl.ds(0, CH)], rbufs[bset][q],
                gsems[bset]).wait()

    fire(0, 0)

    def pair(k, carry):
        for bset in range(2):
            t = 2 * k + bset

            @pl.when(t + 1 < STEPS)
            def _():
                fire(t + 1, 1 - bset)

            gwait(bset)

            @pl.when(t > 0)
            def _():
                pltpu.make_async_copy(
                    obuf, out_hbm.at[pl.ds(0, CH)], osem).wait()

            rbuf = rbufs[bset]

            def row(r, carry2):
                r_idx = jnp.full((16,), r, jnp.int32)
                for q in range(4):
                    for h in range(2):
                        v = rbuf[q][r, pl.ds(16 * h, 16)]
                        plsc.store_scatter(
                            obuf, [r_idx, cols[2 * q + h]], v)
                return carry2

            lax.fori_loop(0, CH, row, 0, unroll=4)

            base = wid * PER_W + t * CH
            pltpu.make_async_copy(
                obuf, out_hbm.at[pl.ds(base, CH)], osem).start()
        return carry

    lax.fori_loop(0, STEPS // 2, pair, 0)

    pltpu.make_async_copy(
        obuf, out_hbm.at[pl.ds(0, CH)], osem).wait()


def kernel(x, scalar, vector_i, vector_j, vector_k):
    xf = x.reshape(-1).astype(jnp.int32)
    tabs = _transpose_tables(
        jnp.transpose(scalar), jnp.transpose(vector_i),
        jnp.transpose(vector_j), jnp.transpose(vector_k))
    out = _emb(xf, *tabs)
    return out.reshape(B, L, DIM, 4)


# speedup vs baseline: 6.9384x; 1.1552x over previous
import functools

import jax
import jax.numpy as jnp
from jax import lax
from jax.experimental import pallas as pl
from jax.experimental.pallas import tpu as pltpu
from jax.experimental.pallas import tpu_sc as plsc

VOCAB = 1000000
DIM = 32
B = 4096
L = 50
NC = 2
NS = 16
NW = NC * NS
CH = B // NW
STEPS = L

TBLK = 4096

_mesh = plsc.VectorSubcoreMesh(core_axis_name="c", subcore_axis_name="s")


def _tr_body(st, it, jt, kt, so, io, jo, ko):
    so[...] = jnp.transpose(st[...])
    io[...] = jnp.transpose(it[...])
    jo[...] = jnp.transpose(jt[...])
    ko[...] = jnp.transpose(kt[...])


_transpose_tables = pl.pallas_call(
    _tr_body,
    grid=(pl.cdiv(VOCAB, TBLK),),
    in_specs=[pl.BlockSpec((DIM, TBLK), lambda i: (0, i))] * 4,
    out_specs=[pl.BlockSpec((TBLK, DIM), lambda i: (i, 0))] * 4,
    out_shape=[jax.ShapeDtypeStruct((VOCAB, DIM), jnp.float32)] * 4,
)


@functools.partial(
    pl.kernel,
    out_type=jax.ShapeDtypeStruct((L, DIM, 4 * B), jnp.float32),
    mesh=_mesh,
    scratch_types=[
        pltpu.VMEM((CH,), jnp.int32),
        [[pltpu.VMEM((CH // 2, DIM), jnp.float32) for _ in range(4)]
         for _ in range(2)],
        pltpu.VMEM((DIM, 4, CH), jnp.float32),
        [pltpu.SemaphoreType.DMA for _ in range(2)],
        pltpu.SemaphoreType.DMA,
    ],
    compiler_params=pltpu.CompilerParams(needs_layout_passes=False),
)
def _emb(x_hbm, s_hbm, vi_hbm, vj_hbm, vk_hbm, out_hbm,
         idxb, rbufs, obuf, gsems, osem):
    wid = lax.axis_index("s") * NC + lax.axis_index("c")
    woff = pl.multiple_of(wid * CH, CH)
    lanes = lax.iota(jnp.int32, 16)
    tables = (s_hbm, vi_hbm, vj_hbm, vk_hbm)
    dvecs = [16 * h + lanes for h in range(2)]
    qvecs = [jnp.full((16,), q, jnp.int32) for q in range(4)]

    HC = CH // 2

    def fire_half(half, bset):
        def grp(g, carry):
            vec = idxb[pl.ds(half * HC + g * 16, 16)]
            for l in range(16):
                v = vec[l]
                for q in range(4):
                    pltpu.make_async_copy(
                        tables[q].at[pl.ds(v, 1)],
                        rbufs[bset][q].at[pl.ds(g * 16 + l, 1)],
                        gsems[bset]).start()
            return carry

        lax.fori_loop(0, HC // 16, grp, 0)

    def gwait(bset):
        for q in range(4):
            pltpu.make_async_copy(
                tables[q].at[pl.ds(0, HC)], rbufs[bset][q],
                gsems[bset]).wait()

    def interleave(half, bset):
        rbuf = rbufs[bset]

        def row(r, carry2):
            s_vec = jnp.full((16,), half * HC + r, jnp.int32)
            for q in range(4):
                for h in range(2):
                    v = rbuf[q][r, pl.ds(16 * h, 16)]
                    plsc.store_scatter(
                        obuf, [dvecs[h], qvecs[q], s_vec], v)
            return carry2

        lax.fori_loop(0, HC, row, 0, unroll=4)

    def owait():
        for q in range(4):
            pltpu.make_async_copy(
                obuf.at[:, q], out_hbm.at[0, :, pl.ds(0, CH)],
                osem).wait()

    pltpu.sync_copy(x_hbm.at[0, pl.ds(woff, CH)], idxb)
    fire_half(0, 0)

    def l_body(t, carry):
        gwait(0)

        @pl.when(t > 0)
        def _():
            owait()

        fire_half(1, 1)
        interleave(0, 0)

        gwait(1)

        @pl.when(t + 1 < STEPS)
        def _():
            pltpu.sync_copy(x_hbm.at[t + 1, pl.ds(woff, CH)], idxb)
            fire_half(0, 0)

        interleave(1, 1)

        for q in range(4):
            pltpu.make_async_copy(
                obuf.at[:, q],
                out_hbm.at[t, :, pl.ds(q * B + woff, CH)],
                osem).start()
        return carry

    lax.fori_loop(0, STEPS, l_body, 0)
    owait()


def kernel(x, scalar, vector_i, vector_j, vector_k):
    xt = jnp.transpose(x).astype(jnp.int32)
    tabs = _transpose_tables(
        jnp.transpose(scalar), jnp.transpose(vector_i),
        jnp.transpose(vector_j), jnp.transpose(vector_k))
    out = _emb(xt, *tabs)
    out = out.reshape(L, DIM, 4, B)
    return jnp.transpose(out, (3, 0, 1, 2))
